# Initial kernel scaffold; baseline (speedup 1.0000x reference)
import jax
import jax.numpy as jnp
from jax.experimental import pallas as pl


def kernel(x, edge_index, W1, b1, W2, b2):
    raise NotImplementedError("write your pallas kernel here")


# speedup vs baseline: 15.3245x; 15.3245x over previous
import functools

import jax
import jax.numpy as jnp
from jax import lax
from jax.experimental import pallas as pl
from jax.experimental.pallas import tpu as pltpu
from jax.experimental.pallas import tpu_sc as plsc


def _proj_body(x_ref, w_ref, b_ref, o_ref):
    o_ref[...] = (
        jnp.dot(x_ref[...], w_ref[...], preferred_element_type=jnp.float32)
        + b_ref[0:1, :]
    )


def _project(x, w, b2d):
    n, d = x.shape
    bn = 1000 if n % 1000 == 0 else n
    grid = n // bn
    return pl.pallas_call(
        _proj_body,
        grid=(grid,),
        in_specs=[
            pl.BlockSpec((bn, d), lambda i: (i, 0)),
            pl.BlockSpec((d, 16), lambda i: (0, 0)),
            pl.BlockSpec((8, 16), lambda i: (0, 0)),
        ],
        out_specs=pl.BlockSpec((bn, 16), lambda i: (i, 0)),
        out_shape=jax.ShapeDtypeStruct((n, 16), jnp.float32),
    )(x, w, b2d)


@functools.cache
def _make_sc_kernel(n_nodes: int, n_edges: int):
    info = plsc.get_sparse_core_info()
    nc, ns, lanes = info.num_cores, info.num_subcores, info.num_lanes
    nw = nc * ns
    epw = n_edges // nw
    assert n_edges % nw == 0 and epw % lanes == 0
    mesh = plsc.VectorSubcoreMesh(core_axis_name="c", subcore_axis_name="s")
    tab_words = n_nodes * 8

    @functools.partial(
        pl.kernel,
        out_type=jax.ShapeDtypeStruct((n_edges,), jnp.float32),
        mesh=mesh,
        scratch_types=[
            pltpu.VMEM((tab_words,), jnp.int32),
            pltpu.VMEM((epw,), jnp.int32),
            pltpu.VMEM((epw,), jnp.int32),
            pltpu.VMEM((epw,), jnp.float32),
            pltpu.VMEM((16,), jnp.float32),
        ],
        compiler_params=pltpu.CompilerParams(needs_layout_passes=False),
    )
    def sc_edge_mlp(tp_hbm, col_hbm, row_hbm, aux_hbm, out_hbm,
                    tab_v, col_v, row_v, out_v, aux_v):
        wid = lax.axis_index("s") * nc + lax.axis_index("c")
        base = wid * epw
        pltpu.sync_copy(tp_hbm, tab_v)
        pltpu.sync_copy(col_hbm.at[pl.ds(base, epw)], col_v)
        pltpu.sync_copy(row_hbm.at[pl.ds(base, epw)], row_v)
        pltpu.sync_copy(aux_hbm, aux_v)
        auxvec = aux_v[...]
        w2 = [auxvec[k] for k in range(8)]
        b2 = auxvec[8]
        himask = jnp.full((lanes,), -65536, jnp.int32)

        def tanh_v(z):
            e = jnp.exp(z + z)
            return 1.0 - 2.0 / (e + 1.0)

        def body(g, carry):
            s = g * lanes
            colv = col_v[pl.ds(s, lanes)]
            rowv = row_v[pl.ds(s, lanes)]
            cb = colv * 8
            rb = rowv * 8 + 4
            acc = jnp.zeros((lanes,), jnp.float32) + b2
            for k in range(4):
                wc = plsc.load_gather(tab_v, [cb + k])
                wr = plsc.load_gather(tab_v, [rb + k])
                z_even = (plsc.bitcast(wc << 16, jnp.float32)
                          + plsc.bitcast(wr << 16, jnp.float32))
                z_odd = (plsc.bitcast(wc & himask, jnp.float32)
                         + plsc.bitcast(wr & himask, jnp.float32))
                acc = acc + w2[2 * k] * tanh_v(z_even)
                acc = acc + w2[2 * k + 1] * tanh_v(z_odd)
            out_v[pl.ds(s, lanes)] = 1.0 / (1.0 + jnp.exp(-acc))
            return carry

        lax.fori_loop(0, epw // lanes, body, 0)
        pltpu.sync_copy(out_v, out_hbm.at[pl.ds(base, epw)])

    return sc_edge_mlp


def kernel(x, edge_index, W1, b1, W2, b2):
    n, d = x.shape
    n_edges = edge_index.shape[1]
    w_cat = jnp.concatenate([W1[:d], W1[d:]], axis=1)
    bias = jnp.concatenate([b1, jnp.zeros((8,), jnp.float32)])
    bias2d = jnp.tile(bias[None, :], (8, 1))
    p = _project(x, w_cat, bias2d)
    packed = lax.bitcast_convert_type(
        p.astype(jnp.bfloat16).reshape(n, 8, 2), jnp.int32
    ).reshape(-1)
    aux = jnp.concatenate([W2[:, 0], b2, jnp.zeros((7,), jnp.float32)])
    out = _make_sc_kernel(n, n_edges)(packed, edge_index[1], edge_index[0], aux)
    return out[:, None]


# speedup vs baseline: 18.7245x; 1.2219x over previous
import functools

import jax
import jax.numpy as jnp
from jax import lax
from jax.experimental import pallas as pl
from jax.experimental.pallas import tpu as pltpu
from jax.experimental.pallas import tpu_sc as plsc


def _rtne_bf16_bits(p):
    u = lax.bitcast_convert_type(p, jnp.uint32)
    r = u + jnp.uint32(0x7FFF) + ((u >> 16) & jnp.uint32(1))
    return r & jnp.uint32(0xFFFF0000)


def _proj_pack_body(x_ref, we_ref, wo_ref, be_ref, bo_ref, o_ref):
    pe = (
        jnp.dot(x_ref[...], we_ref[...], preferred_element_type=jnp.float32)
        + be_ref[0:1, :]
    )
    po = (
        jnp.dot(x_ref[...], wo_ref[...], preferred_element_type=jnp.float32)
        + bo_ref[0:1, :]
    )
    ue = _rtne_bf16_bits(pe)
    uo = _rtne_bf16_bits(po)
    o_ref[...] = lax.bitcast_convert_type((ue >> 16) | uo, jnp.int32)


def _project_pack(x, we, wo, be, bo):
    n, d = x.shape
    bn = 2000 if n % 2000 == 0 else n
    grid = n // bn
    return pl.pallas_call(
        _proj_pack_body,
        grid=(grid,),
        in_specs=[
            pl.BlockSpec((bn, d), lambda i: (i, 0)),
            pl.BlockSpec((d, 8), lambda i: (0, 0)),
            pl.BlockSpec((d, 8), lambda i: (0, 0)),
            pl.BlockSpec((8, 8), lambda i: (0, 0)),
            pl.BlockSpec((8, 8), lambda i: (0, 0)),
        ],
        out_specs=pl.BlockSpec((bn, 8), lambda i: (i, 0)),
        out_shape=jax.ShapeDtypeStruct((n, 8), jnp.int32),
    )(x, we, wo, be, bo)


@functools.cache
def _make_sc_kernel(n_nodes: int, n_edges: int):
    info = plsc.get_sparse_core_info()
    nc, ns, lanes = info.num_cores, info.num_subcores, info.num_lanes
    nw = nc * ns
    epw = n_edges // nw
    assert n_edges % nw == 0 and epw % lanes == 0 and epw % 8 == 0
    mesh = plsc.VectorSubcoreMesh(core_axis_name="c", subcore_axis_name="s")
    tab_words = n_nodes * 8

    @functools.partial(
        pl.kernel,
        out_type=jax.ShapeDtypeStruct((n_edges,), jnp.float32),
        mesh=mesh,
        scratch_types=[
            pltpu.VMEM((tab_words,), jnp.int32),
            pltpu.VMEM((epw,), jnp.int32),
            pltpu.VMEM((epw,), jnp.int32),
            pltpu.VMEM((epw,), jnp.float32),
            pltpu.VMEM((16,), jnp.float32),
        ],
        compiler_params=pltpu.CompilerParams(needs_layout_passes=False),
    )
    def sc_edge_mlp(tp_hbm, ei_hbm, aux_hbm, out_hbm,
                    tab_v, col_v, row_v, out_v, aux_v):
        wid = lax.axis_index("s") * nc + lax.axis_index("c")
        base = wid * epw
        pltpu.sync_copy(tp_hbm, tab_v)
        pltpu.sync_copy(ei_hbm.at[pl.ds(n_edges + base, epw)], col_v)
        pltpu.sync_copy(ei_hbm.at[pl.ds(base, epw)], row_v)
        pltpu.sync_copy(aux_hbm, aux_v)
        auxvec = aux_v[...]
        dk = [auxvec[k] for k in range(8)]
        neg_c = auxvec[8]
        himask = jnp.full((lanes,), -65536, jnp.int32)

        def body(g, carry):
            s = g * lanes
            colv = col_v[pl.ds(s, lanes)]
            rowv = row_v[pl.ds(s, lanes)]
            cb = colv << 3
            rb = (rowv << 3) + 4
            neg = jnp.zeros((lanes,), jnp.float32) + neg_c
            for k in range(4):
                wc = plsc.load_gather(tab_v, [cb + k])
                wr = plsc.load_gather(tab_v, [rb + k])
                z_even = (plsc.bitcast(wc << 16, jnp.float32)
                          + plsc.bitcast(wr << 16, jnp.float32))
                z_odd = (plsc.bitcast(wc & himask, jnp.float32)
                         + plsc.bitcast(wr & himask, jnp.float32))
                neg = neg + dk[2 * k] / (jnp.exp(z_even) + 1.0)
                neg = neg + dk[2 * k + 1] / (jnp.exp(z_odd) + 1.0)
            out_v[pl.ds(s, lanes)] = 1.0 / (1.0 + jnp.exp(neg))
            return carry

        lax.fori_loop(0, epw // lanes, body, 0)
        pltpu.sync_copy(out_v, out_hbm.at[pl.ds(base, epw)])

    return sc_edge_mlp


def kernel(x, edge_index, W1, b1, W2, b2):
    n, d = x.shape
    n_edges = edge_index.shape[1]
    we = 2.0 * jnp.concatenate([W1[:d, 0::2], W1[d:, 0::2]], axis=1)
    wo = 2.0 * jnp.concatenate([W1[:d, 1::2], W1[d:, 1::2]], axis=1)
    zeros4 = jnp.zeros((4,), jnp.float32)
    be = jnp.tile(jnp.concatenate([2.0 * b1[0::2], zeros4])[None, :], (8, 1))
    bo = jnp.tile(jnp.concatenate([2.0 * b1[1::2], zeros4])[None, :], (8, 1))
    packed = _project_pack(x, we, wo, be, bo).reshape(-1)
    w2v = W2[:, 0]
    aux = jnp.concatenate(
        [2.0 * w2v, -(b2 + jnp.sum(w2v)), jnp.zeros((7,), jnp.float32)]
    )
    out = _make_sc_kernel(n, n_edges)(packed, edge_index.reshape(-1), aux)
    return out[:, None]


# speedup vs baseline: 23.1752x; 1.2377x over previous
import functools

import jax
import jax.numpy as jnp
from jax import lax
from jax.experimental import pallas as pl
from jax.experimental.pallas import tpu as pltpu
from jax.experimental.pallas import tpu_sc as plsc


def _rtne_bf16_bits(p):
    u = lax.bitcast_convert_type(p, jnp.uint32)
    r = u + jnp.uint32(0x7FFF) + ((u >> 16) & jnp.uint32(1))
    return r & jnp.uint32(0xFFFF0000)


def _proj_pack_body(x_ref, we_ref, wo_ref, be_ref, bo_ref, o_ref):
    pe = (
        jnp.dot(x_ref[...], we_ref[...], preferred_element_type=jnp.float32)
        + be_ref[0:1, :]
    )
    po = (
        jnp.dot(x_ref[...], wo_ref[...], preferred_element_type=jnp.float32)
        + bo_ref[0:1, :]
    )
    ue = _rtne_bf16_bits(pe)
    uo = _rtne_bf16_bits(po)
    o_ref[...] = lax.bitcast_convert_type((ue >> 16) | uo, jnp.int32)


def _project_pack(x, we, wo, be, bo):
    n, d = x.shape
    bn = 2000 if n % 2000 == 0 else n
    grid = n // bn
    return pl.pallas_call(
        _proj_pack_body,
        grid=(grid,),
        in_specs=[
            pl.BlockSpec((bn, d), lambda i: (i, 0)),
            pl.BlockSpec((d, 8), lambda i: (0, 0)),
            pl.BlockSpec((d, 8), lambda i: (0, 0)),
            pl.BlockSpec((8, 8), lambda i: (0, 0)),
            pl.BlockSpec((8, 8), lambda i: (0, 0)),
        ],
        out_specs=pl.BlockSpec((bn, 8), lambda i: (i, 0)),
        out_shape=jax.ShapeDtypeStruct((n, 8), jnp.int32),
    )(x, we, wo, be, bo)


@functools.cache
def _make_sc_kernel(n_nodes: int, n_edges: int):
    info = plsc.get_sparse_core_info()
    nc, ns, lanes = info.num_cores, info.num_subcores, info.num_lanes
    nw = nc * ns
    epw = n_edges // nw
    assert n_edges % nw == 0 and epw % lanes == 0 and epw % 8 == 0
    mesh = plsc.VectorSubcoreMesh(core_axis_name="c", subcore_axis_name="s")
    tab_words = n_nodes * 8

    @functools.partial(
        pl.kernel,
        out_type=jax.ShapeDtypeStruct((n_edges,), jnp.float32),
        mesh=mesh,
        scratch_types=[
            pltpu.VMEM((tab_words,), jnp.int32),
            pltpu.VMEM((epw,), jnp.int32),
            pltpu.VMEM((epw,), jnp.int32),
            pltpu.VMEM((epw,), jnp.float32),
            pltpu.VMEM((16,), jnp.float32),
        ],
        compiler_params=pltpu.CompilerParams(needs_layout_passes=False),
    )
    def sc_edge_mlp(tp_hbm, ei_hbm, aux_hbm, out_hbm,
                    tab_v, col_v, row_v, out_v, aux_v):
        wid = lax.axis_index("s") * nc + lax.axis_index("c")
        base = wid * epw
        pltpu.sync_copy(tp_hbm, tab_v)
        pltpu.sync_copy(ei_hbm.at[pl.ds(n_edges + base, epw)], col_v)
        pltpu.sync_copy(ei_hbm.at[pl.ds(base, epw)], row_v)
        pltpu.sync_copy(aux_hbm, aux_v)
        auxvec = aux_v[...]
        dk = [auxvec[k] for k in range(8)]
        neg_c = auxvec[8]
        himask = jnp.full((lanes,), -65536, jnp.int32)

        @plsc.parallel_loop(0, epw, step=lanes, unroll=2)
        def body(s):
            colv = col_v[pl.ds(s, lanes)]
            rowv = row_v[pl.ds(s, lanes)]
            cb = colv << 3
            rb = (rowv << 3) + 4
            neg = jnp.zeros((lanes,), jnp.float32) + neg_c
            for k in range(4):
                wc = plsc.load_gather(tab_v, [cb + k])
                wr = plsc.load_gather(tab_v, [rb + k])
                z_even = (plsc.bitcast(wc << 16, jnp.float32)
                          + plsc.bitcast(wr << 16, jnp.float32))
                z_odd = (plsc.bitcast(wc & himask, jnp.float32)
                         + plsc.bitcast(wr & himask, jnp.float32))
                ae = jnp.exp(z_even) + 1.0
                ao = jnp.exp(z_odd) + 1.0
                neg = neg + (dk[2 * k] * ao + dk[2 * k + 1] * ae) / (ae * ao)
            out_v[pl.ds(s, lanes)] = 1.0 / (1.0 + jnp.exp(neg))
        pltpu.sync_copy(out_v, out_hbm.at[pl.ds(base, epw)])

    return sc_edge_mlp


def kernel(x, edge_index, W1, b1, W2, b2):
    n, d = x.shape
    n_edges = edge_index.shape[1]
    we = 2.0 * jnp.concatenate([W1[:d, 0::2], W1[d:, 0::2]], axis=1)
    wo = 2.0 * jnp.concatenate([W1[:d, 1::2], W1[d:, 1::2]], axis=1)
    zeros4 = jnp.zeros((4,), jnp.float32)
    be = jnp.tile(jnp.concatenate([2.0 * b1[0::2], zeros4])[None, :], (8, 1))
    bo = jnp.tile(jnp.concatenate([2.0 * b1[1::2], zeros4])[None, :], (8, 1))
    packed = _project_pack(x, we, wo, be, bo).reshape(-1)
    w2v = W2[:, 0]
    aux = jnp.concatenate(
        [2.0 * w2v, -(b2 + jnp.sum(w2v)), jnp.zeros((7,), jnp.float32)]
    )
    out = _make_sc_kernel(n, n_edges)(packed, edge_index.reshape(-1), aux)
    return out[:, None]
